# Initial kernel scaffold; baseline (speedup 1.0000x reference)
import jax
import jax.numpy as jnp
from jax.experimental import pallas as pl


def kernel(input_sentence, word_embedding_weight):
    raise NotImplementedError("write your pallas kernel here")


# speedup vs baseline: 1.8447x; 1.8447x over previous
import functools

import jax
import jax.numpy as jnp
from jax import lax
from jax.experimental import pallas as pl
from jax.experimental.pallas import tpu as pltpu
from jax.experimental.pallas import tpu_sc as plsc

_IDX_MINOR = 128


@functools.lru_cache(maxsize=None)
def _build(B, V, D, chunk):
  NC, NS = 2, 16
  NW = NC * NS
  n_stream = chunk // _IDX_MINOR
  b_per_w = B // NW
  n_chunk = b_per_w // chunk
  rows_per_w_idx = b_per_w // _IDX_MINOR

  mesh = plsc.VectorSubcoreMesh(core_axis_name="c", subcore_axis_name="s")

  @functools.partial(
      pl.kernel,
      mesh=mesh,
      compiler_params=pltpu.CompilerParams(use_tc_tiling_on_sc=False),
      out_type=jax.ShapeDtypeStruct((B, D), jnp.float32),
      scratch_types=[
          pltpu.VMEM((n_stream, _IDX_MINOR), jnp.int32),
          pltpu.VMEM((chunk, D), jnp.float32),
          pltpu.SemaphoreType.DMA,
      ],
  )
  def gather_kernel(idx_hbm, table_hbm, out_hbm, idx_v, rows_v, sem):
    wid = lax.axis_index("s") * NC + lax.axis_index("c")
    row_base_w = wid * rows_per_w_idx
    out_base_w = wid * b_per_w

    def body(g, carry):
      pltpu.sync_copy(idx_hbm.at[pl.ds(row_base_w + g * n_stream, n_stream)],
                      idx_v)
      handles = []
      for j in range(n_stream):
        handles.append(
            pltpu.async_copy(table_hbm.at[idx_v.at[j]],
                             rows_v.at[pl.ds(j * _IDX_MINOR, _IDX_MINOR)],
                             sem))
      for h in handles:
        h.wait()
      pltpu.sync_copy(rows_v, out_hbm.at[pl.ds(out_base_w + g * chunk, chunk)])
      return carry

    lax.fori_loop(0, n_chunk, body, 0)

  return gather_kernel


def kernel(input_sentence, word_embedding_weight):
  S, W = input_sentence.shape
  V, D = word_embedding_weight.shape
  B = S * W
  idx2d = input_sentence.reshape(B // _IDX_MINOR, _IDX_MINOR).astype(jnp.int32)
  fn = _build(B, V, D, 1024)
  out = fn(idx2d, word_embedding_weight)
  return out.reshape(S, W, D)


# speedup vs baseline: 1.8644x; 1.0107x over previous
import functools

import jax
import jax.numpy as jnp
from jax import lax
from jax.experimental import pallas as pl
from jax.experimental.pallas import tpu as pltpu
from jax.experimental.pallas import tpu_sc as plsc

_IDX_MINOR = 128
_NBUF = 2


@functools.lru_cache(maxsize=None)
def _build(B, V, D, chunk):
  NC, NS = 2, 16
  NW = NC * NS
  n_stream = chunk // _IDX_MINOR
  b_per_w = B // NW
  n_chunk = b_per_w // chunk
  idx_rows_w = b_per_w // _IDX_MINOR
  assert n_chunk % _NBUF == 0

  mesh = plsc.VectorSubcoreMesh(core_axis_name="c", subcore_axis_name="s")

  @functools.partial(
      pl.kernel,
      mesh=mesh,
      compiler_params=pltpu.CompilerParams(use_tc_tiling_on_sc=False),
      out_type=jax.ShapeDtypeStruct((B, D), jnp.float32),
      scratch_types=[
          pltpu.VMEM((idx_rows_w, _IDX_MINOR), jnp.int32),
          pltpu.VMEM((_NBUF * chunk, D), jnp.float32),
          pltpu.SemaphoreType.DMA((_NBUF,)),
          pltpu.SemaphoreType.DMA((_NBUF,)),
      ],
  )
  def gather_kernel(idx_hbm, table_hbm, out_hbm, idx_v, rows_v, gsem, osem):
    wid = lax.axis_index("s") * NC + lax.axis_index("c")
    out_base_w = wid * b_per_w

    pltpu.sync_copy(idx_hbm.at[pl.ds(wid * idx_rows_w, idx_rows_w)], idx_v)

    def fire_gathers(g, b):
      for j in range(n_stream):
        pltpu.async_copy(
            table_hbm.at[idx_v.at[g * n_stream + j]],
            rows_v.at[pl.ds(b * chunk + j * _IDX_MINOR, _IDX_MINOR)],
            gsem.at[b])

    def drain_gathers(b):
      pltpu.make_async_copy(table_hbm.at[pl.ds(0, chunk)],
                            rows_v.at[pl.ds(b * chunk, chunk)],
                            gsem.at[b]).wait()

    def wait_outwrite(b):
      pltpu.make_async_copy(rows_v.at[pl.ds(b * chunk, chunk)],
                            out_hbm.at[pl.ds(out_base_w, chunk)],
                            osem.at[b]).wait()

    for b in range(_NBUF):
      fire_gathers(b, b)

    def body(t, carry):
      for b in range(_NBUF):
        g = _NBUF * t + b
        drain_gathers(b)
        pltpu.async_copy(rows_v.at[pl.ds(b * chunk, chunk)],
                         out_hbm.at[pl.ds(out_base_w + g * chunk, chunk)],
                         osem.at[b])
      for b in range(_NBUF):
        g_next = _NBUF * t + b + _NBUF

        @pl.when(g_next < n_chunk)
        def _():
          wait_outwrite(b)
          fire_gathers(g_next, b)

      return carry

    lax.fori_loop(0, n_chunk // _NBUF, body, 0)
    for b in range(_NBUF):
      wait_outwrite(b)

  return gather_kernel


def kernel(input_sentence, word_embedding_weight):
  S, W = input_sentence.shape
  V, D = word_embedding_weight.shape
  B = S * W
  idx2d = input_sentence.reshape(B // _IDX_MINOR, _IDX_MINOR).astype(jnp.int32)
  fn = _build(B, V, D, 640)
  out = fn(idx2d, word_embedding_weight)
  return out.reshape(S, W, D)
